# baseline (speedup 1.0000x reference)
import jax
import jax.numpy as jnp
from jax.experimental import pallas as pl

_BM = 200


def _support_kernel(feature_ref, w_ref, out_ref):
    acc = jnp.dot(feature_ref[...], w_ref[...], preferred_element_type=jnp.float32)
    out_ref[...] = jnp.maximum(acc, 0.0)


def _agg_kernel(adj_ref, support_ref, support_diag_ref, b_ref, out_ref):
    acc = jnp.dot(adj_ref[...], support_ref[...], preferred_element_type=jnp.float32)
    out_ref[...] = acc + support_diag_ref[...] + b_ref[...]


def kernel(feature, adj, W, b):
    n, _ = feature.shape
    d = W.shape[1]

    support = pl.pallas_call(
        _support_kernel,
        out_shape=jax.ShapeDtypeStruct((n, d), jnp.float32),
    )(feature, W)

    bm = _BM
    out = pl.pallas_call(
        _agg_kernel,
        grid=(n // bm,),
        in_specs=[
            pl.BlockSpec((bm, n), lambda i: (i, 0)),
            pl.BlockSpec((n, d), lambda i: (0, 0)),
            pl.BlockSpec((bm, d), lambda i: (i, 0)),
            pl.BlockSpec((1, d), lambda i: (0, 0)),
        ],
        out_specs=pl.BlockSpec((bm, d), lambda i: (i, 0)),
        out_shape=jax.ShapeDtypeStruct((n, d), jnp.float32),
    )(adj, support, support, b.reshape(1, d))
    return out
